# baseline (speedup 1.0000x reference)
import functools

import jax
import jax.numpy as jnp
from jax import lax
from jax.experimental import pallas as pl
from jax.experimental.pallas import tpu as pltpu
from jax.experimental.pallas import tpu_sc as plsc

VOCAB = 1000
BATCH = 1024
SEQ = 50
N = BATCH * SEQ * VOCAB
NC = 2
NS = 16
NW = NC * NS
EPW = N // NW
ZCH = 50000
NFILL = EPW // ZCH
SLABS_PER_SC = SEQ // NC
SLAB = VOCAB * BATCH

_mesh = plsc.VectorSubcoreMesh(core_axis_name="c", subcore_axis_name="s")


@functools.partial(
    pl.kernel,
    mesh=_mesh,
    out_type=jax.ShapeDtypeStruct((N,), jnp.float32),
    compiler_params=pltpu.CompilerParams(
        needs_layout_passes=False, use_tc_tiling_on_sc=False
    ),
    scratch_types=[
        pltpu.VMEM((ZCH,), jnp.float32),
        pltpu.VMEM((ZCH,), jnp.float32),
        pltpu.VMEM((BATCH,), jnp.int32),
        pltpu.VMEM((BATCH,), jnp.float32),
        pltpu.VMEM((16, 128), jnp.int32),
        pltpu.VMEM((16, 128), jnp.float32),
        pltpu.SemaphoreType.DMA,
        pltpu.SemaphoreType.DMA,
    ],
)
def _onehot_sc(idxT_hbm, mskT_hbm, zeros_hbm, out_hbm,
               zbuf0, zbuf1, sidx, smsk, abuf, vbuf, fsem, ssem):
    cid = lax.axis_index("c")
    sid = lax.axis_index("s")
    wid = cid * NS + sid
    base = wid * EPW

    pltpu.sync_copy(zeros_hbm, zbuf0)
    pltpu.sync_copy(zeros_hbm, zbuf1)

    def fstep(i, carry):
        pltpu.make_async_copy(
            zbuf0, out_hbm.at[pl.ds(base + i * 2 * ZCH, ZCH)], fsem
        ).start()
        pltpu.make_async_copy(
            zbuf1, out_hbm.at[pl.ds(base + (i * 2 + 1) * ZCH, ZCH)], fsem
        ).start()

        @pl.when(i >= 1)
        def _():
            pltpu.make_async_copy(
                zbuf0, out_hbm.at[pl.ds(base + (i - 1) * 2 * ZCH, ZCH)], fsem
            ).wait()
            pltpu.make_async_copy(
                zbuf1, out_hbm.at[pl.ds(base + ((i - 1) * 2 + 1) * ZCH, ZCH)],
                fsem
            ).wait()
        return carry

    lax.fori_loop(0, NFILL // 2, fstep, 0)

    lo = (sid * SLABS_PER_SC + NS - 1) // NS
    hi = ((sid + 1) * SLABS_PER_SC + NS - 1) // NS
    lane = lax.iota(jnp.int32, 16)

    def prep_slab(sub, carry):
        tg = cid * SLABS_PER_SC + sub
        pltpu.sync_copy(idxT_hbm.at[pl.ds(tg * BATCH, BATCH)], sidx)
        pltpu.sync_copy(mskT_hbm.at[pl.ds(tg * BATCH, BATCH)], smsk)
        tbase = tg * SLAB
        rbase = (sub - lo) * 8
        for k in range(BATCH // 16):
            iv = sidx[pl.ds(k * 16, 16)]
            mv = smsk[pl.ds(k * 16, 16)]
            va = ((iv >> 3) << 13) + ((iv & 7) << 7)
            addr = va + (tbase + (k // 8) * 1024 + (k % 8) * 16) + lane
            row = jnp.broadcast_to(rbase + k // 8, (16,)).astype(jnp.int32)
            col = (k % 8) * 16 + lane
            plsc.store_scatter(abuf, [row, col], addr)
            plsc.store_scatter(vbuf, [row, col], mv)
        return carry

    lax.fori_loop(lo, hi, prep_slab, 0)

    last = NFILL // 2 - 1
    pltpu.make_async_copy(
        zbuf0, out_hbm.at[pl.ds(base + last * 2 * ZCH, ZCH)], fsem
    ).wait()
    pltpu.make_async_copy(
        zbuf1, out_hbm.at[pl.ds(base + (last * 2 + 1) * ZCH, ZCH)], fsem
    ).wait()

    plsc.subcore_barrier()

    def fire_slab(sub, carry):
        rbase = (sub - lo) * 8
        for j in range(8):
            pltpu.make_async_copy(
                vbuf.at[rbase + j], out_hbm.at[abuf.at[rbase + j]], ssem
            ).start()
        return carry

    lax.fori_loop(lo, hi, fire_slab, 0)

    def drain_slab(sub, carry):
        rbase = (sub - lo) * 8
        for j in range(8):
            pltpu.make_async_copy(
                vbuf.at[rbase + j], out_hbm.at[abuf.at[rbase + j]], ssem
            ).wait()
        return carry

    lax.fori_loop(lo, hi, drain_slab, 0)


def kernel(array, mask):
    idxT = array.astype(jnp.int32).T.reshape(SEQ * BATCH)
    mskT = mask.astype(jnp.float32).T.reshape(SEQ * BATCH)
    zeros = jnp.zeros((ZCH,), jnp.float32)
    out = _onehot_sc(idxT, mskT, zeros)
    out5 = out.reshape(SEQ, VOCAB // 8, 8, 8, 128)
    return out5.transpose(2, 4, 0, 1, 3).reshape(BATCH, SEQ, VOCAB)


# speedup vs baseline: 1.0310x; 1.0310x over previous
import functools

import jax
import jax.numpy as jnp
from jax import lax
from jax.experimental import pallas as pl
from jax.experimental.pallas import tpu as pltpu
from jax.experimental.pallas import tpu_sc as plsc

VOCAB = 1000
BATCH = 1024
SEQ = 50
N = BATCH * SEQ * VOCAB
NC = 2
NS = 16
NW = NC * NS
EPW = N // NW
ZCH = 64000
NFILL = EPW // ZCH
SLABS_PER_SC = SEQ // NC
SLAB = VOCAB * BATCH

_mesh = plsc.VectorSubcoreMesh(core_axis_name="c", subcore_axis_name="s")


@functools.partial(
    pl.kernel,
    mesh=_mesh,
    out_type=jax.ShapeDtypeStruct((N,), jnp.float32),
    compiler_params=pltpu.CompilerParams(
        needs_layout_passes=False, use_tc_tiling_on_sc=False
    ),
    scratch_types=[
        pltpu.VMEM((ZCH,), jnp.float32),
        pltpu.VMEM((BATCH,), jnp.int32),
        pltpu.VMEM((BATCH,), jnp.float32),
        pltpu.VMEM((8, 128), jnp.int32),
        pltpu.VMEM((8, 128), jnp.float32),
        pltpu.SemaphoreType.DMA,
        pltpu.SemaphoreType.DMA,
    ],
)
def _onehot_sc(idxT_hbm, mskT_hbm, zeros_hbm, out_hbm,
               zbuf, sidx, smsk, abuf, vbuf, fsem, ssem):
    cid = lax.axis_index("c")
    sid = lax.axis_index("s")
    wid = cid * NS + sid
    base = wid * EPW

    pltpu.sync_copy(zeros_hbm, zbuf)

    def fstart(i, carry):
        pltpu.make_async_copy(
            zbuf, out_hbm.at[pl.ds(base + i * ZCH, ZCH)], fsem
        ).start()
        return carry

    lax.fori_loop(0, NFILL, fstart, 0)

    def fwait(i, carry):
        pltpu.make_async_copy(
            zbuf, out_hbm.at[pl.ds(base + i * ZCH, ZCH)], fsem
        ).wait()
        return carry

    lax.fori_loop(0, NFILL, fwait, 0)

    plsc.subcore_barrier()

    lane = lax.iota(jnp.int32, 16)

    def do_slab(sub, carry):
        @pl.when(sub * NS // SLABS_PER_SC == sid)
        def _():
            tg = cid * SLABS_PER_SC + sub
            pltpu.sync_copy(idxT_hbm.at[pl.ds(tg * BATCH, BATCH)], sidx)
            pltpu.sync_copy(mskT_hbm.at[pl.ds(tg * BATCH, BATCH)], smsk)
            tbase = tg * SLAB
            for k in range(BATCH // 16):
                iv = sidx[pl.ds(k * 16, 16)]
                mv = smsk[pl.ds(k * 16, 16)]
                va = ((iv >> 3) << 13) + ((iv & 7) << 7)
                addr = va + (tbase + (k // 8) * 1024 + (k % 8) * 16) + lane
                row = jnp.full((16,), k // 8, jnp.int32)
                col = (k % 8) * 16 + lane
                plsc.store_scatter(abuf, [row, col], addr)
                plsc.store_scatter(vbuf, [row, col], mv)
            for j in range(8):
                pltpu.make_async_copy(
                    vbuf.at[j], out_hbm.at[abuf.at[j]], ssem
                ).start()
            for j in range(8):
                pltpu.make_async_copy(
                    vbuf.at[j], out_hbm.at[abuf.at[j]], ssem
                ).wait()
        return carry

    lax.fori_loop(0, SLABS_PER_SC, do_slab, 0)


def kernel(array, mask):
    idxT = array.astype(jnp.int32).T.reshape(SEQ * BATCH)
    mskT = mask.astype(jnp.float32).T.reshape(SEQ * BATCH)
    zeros = jnp.zeros((ZCH,), jnp.float32)
    out = _onehot_sc(idxT, mskT, zeros)
    out5 = out.reshape(SEQ, VOCAB // 8, 8, 8, 128)
    return out5.transpose(2, 4, 0, 1, 3).reshape(BATCH, SEQ, VOCAB)
